# baseline (speedup 1.0000x reference)
import functools

import jax
import jax.numpy as jnp
from jax import lax
from jax.experimental import pallas as pl
from jax.experimental.pallas import tpu as pltpu
from jax.experimental.pallas import tpu_sc as plsc

B = 16384
D = 64
NC = 2
NS = 16
NW = NC * NS
BPW = B // NW
L = 16
CH = D // L
GROUPS = BPW // L
HALF = BPW // 2
GPH = GROUPS // 2


def _dist_mult_body(hidx_hbm, ridx_hbm, tidx_hbm, ent_hbm, rel_hbm,
                    score_hbm, part_hbm,
                    hraw_v, rraw_v, traw_v, hpix_v, rpix_v, tpix_v,
                    hbuf, tbuf, rbuf, score_v, part_v, sem0, sem1):
    wid = lax.axis_index("s") * NC + lax.axis_index("c")
    base = wid * BPW

    pltpu.sync_copy(hidx_hbm.at[pl.ds(base, BPW)], hraw_v)
    pltpu.sync_copy(tidx_hbm.at[pl.ds(base, BPW)], traw_v)
    pltpu.sync_copy(ridx_hbm.at[pl.ds(base, BPW)], rraw_v)

    for raw, pix in ((hraw_v, hpix_v), (traw_v, tpix_v), (rraw_v, rpix_v)):
        for q in range(4):
            for k in range(8):
                v = raw[pl.ds(q * 128 + k * L, L)]
                pix[q, pl.ds(k * L, L)] = v - jnp.where(
                    v >= H, jnp.int32(H), jnp.int32(0))

    def fire(q, sem):
        dst = pl.ds(lax.rem(q, 2) * 128, 128)
        pltpu.async_copy(ent_hbm.at[hpix_v.at[q]], hbuf.at[dst], sem)
        pltpu.async_copy(ent_hbm.at[tpix_v.at[q]], tbuf.at[dst], sem)
        pltpu.async_copy(rel_hbm.at[rpix_v.at[q]], rbuf.at[dst], sem)

    def drain(sem):
        for buf in (hbuf, tbuf, rbuf):
            pltpu.make_async_copy(
                ent_hbm.at[pl.ds(0, 128)],
                buf.at[pl.ds(0, 128)], sem).wait()

    lane = jnp.arange(L, dtype=jnp.int32)
    masks = [lane == i for i in range(L)]

    GPQ = 8
    fire(0, sem0)

    def step(g, sq_acc):
        @pl.when(lax.rem(g, GPQ) == 0)
        def _():
            q = g // GPQ
            par = lax.rem(q, 2)

            @pl.when(q < 3)
            def _():
                @pl.when(par == 0)
                def _():
                    fire(q + 1, sem1)

                @pl.when(par == 1)
                def _():
                    fire(q + 1, sem0)

            @pl.when(par == 0)
            def _():
                drain(sem0)

            @pl.when(par == 1)
            def _():
                drain(sem1)

        hv = hraw_v[pl.ds(g * L, L)]
        tv = traw_v[pl.ds(g * L, L)]
        rv = rraw_v[pl.ds(g * L, L)]
        grow = lax.rem(g, 2 * GPQ) * L
        acc = jnp.zeros((L,), jnp.float32)
        for i in range(L):
            rowv = jnp.full((L,), grow + i, jnp.int32)
            hc = jnp.where(hv[i] >= H, D, 0) + lane
            tc = jnp.where(tv[i] >= H, D, 0) + lane
            rc = jnp.where(rv[i] >= H, D, 0) + lane
            racc = jnp.zeros((L,), jnp.float32)
            for k in range(CH):
                h = plsc.load_gather(hbuf, [rowv, hc + k * L])
                t = plsc.load_gather(tbuf, [rowv, tc + k * L])
                r = plsc.load_gather(rbuf, [rowv, rc + k * L])
                racc = racc + h * t * r
                sq_acc = sq_acc + h * h + t * t + r * r
            acc = jnp.where(masks[i], jnp.sum(racc), acc)
        score_v[pl.ds(g * L, L)] = -acc
        return sq_acc

    sq_acc = lax.fori_loop(0, GROUPS, step, jnp.zeros((L,), jnp.float32))

    part_v[...] = sq_acc
    pltpu.sync_copy(score_v, score_hbm.at[pl.ds(base, BPW)])
    pltpu.sync_copy(part_v, part_hbm.at[wid])


@jax.jit
def _dist_mult_sc(h_idx, r_idx, t_idx, ent_p, rel_p):
    mesh = plsc.VectorSubcoreMesh(core_axis_name="c", subcore_axis_name="s")
    call = functools.partial(
        pl.kernel,
        mesh=mesh,
        compiler_params=pltpu.CompilerParams(
            needs_layout_passes=False, use_tc_tiling_on_sc=True),
        out_type=[
            jax.ShapeDtypeStruct((B,), jnp.float32),
            jax.ShapeDtypeStruct((NW, L), jnp.float32),
        ],
        scratch_types=[
            pltpu.VMEM((BPW,), jnp.int32),
            pltpu.VMEM((BPW,), jnp.int32),
            pltpu.VMEM((BPW,), jnp.int32),
            pltpu.VMEM((4, 128), jnp.int32),
            pltpu.VMEM((4, 128), jnp.int32),
            pltpu.VMEM((4, 128), jnp.int32),
            pltpu.VMEM((HALF, 2 * D), jnp.float32),
            pltpu.VMEM((HALF, 2 * D), jnp.float32),
            pltpu.VMEM((HALF, 2 * D), jnp.float32),
            pltpu.VMEM((BPW,), jnp.float32),
            pltpu.VMEM((L,), jnp.float32),
            pltpu.SemaphoreType.DMA,
            pltpu.SemaphoreType.DMA,
        ],
    )(_dist_mult_body)
    return call(h_idx, r_idx, t_idx, ent_p, rel_p)


NE = 1000000
CW = 20480
NCHUNK = 25
H = NCHUNK * CW


def _repack_body(lo_ref, hi_ref, out_ref):
    eye = jnp.eye(D, dtype=jnp.float32)
    out_ref[:, 0:D] = jax.lax.dot_general(
        lo_ref[...], eye, (((0,), (0,)), ((), ())),
        preferred_element_type=jnp.float32)
    out_ref[:, D:2 * D] = jax.lax.dot_general(
        hi_ref[...], eye, (((0,), (0,)), ((), ())),
        preferred_element_type=jnp.float32)


def _repack(table_t):
    last_blk = (NE - 1) // CW
    return pl.pallas_call(
        _repack_body,
        grid=(NCHUNK,),
        in_specs=[
            pl.BlockSpec((D, CW), lambda i: (0, i)),
            pl.BlockSpec((D, CW),
                         lambda i: (0, jnp.minimum(i + NCHUNK, last_blk))),
        ],
        out_specs=pl.BlockSpec((CW, 2 * D), lambda i: (i, 0)),
        out_shape=jax.ShapeDtypeStruct((H, 2 * D), jnp.float32),
    )(table_t, table_t)


def kernel(batch_input, ent_embeddings, rel_embeddings):
    bi = batch_input.astype(jnp.int32)
    h_idx = bi[:, 0]
    r_idx = bi[:, 1]
    t_idx = bi[:, 2]
    ent_p = _repack(ent_embeddings.T)
    rel_p = _repack(rel_embeddings.T)
    score, part = _dist_mult_sc(h_idx, r_idx, t_idx, ent_p, rel_p)
    regul = jnp.sum(part) / jnp.float32(B * D)
    return (score, regul)


# speedup vs baseline: 1.0100x; 1.0100x over previous
import functools

import jax
import jax.numpy as jnp
from jax import lax
from jax.experimental import pallas as pl
from jax.experimental.pallas import tpu as pltpu
from jax.experimental.pallas import tpu_sc as plsc

B = 16384
D = 64
NC = 2
NS = 16
NW = NC * NS
BPW = B // NW
L = 16
CH = D // L
GROUPS = BPW // L
HALF = BPW // 2
GPH = GROUPS // 2


def _dist_mult_body(hidx_hbm, ridx_hbm, tidx_hbm, ent_hbm, rel_hbm,
                    score_hbm, part_hbm,
                    hraw_v, rraw_v, traw_v, hpix_v, rpix_v, tpix_v,
                    hbuf, tbuf, rbuf, score_v, part_v, sem0, sem1):
    wid = lax.axis_index("s") * NC + lax.axis_index("c")
    base = wid * BPW

    pltpu.sync_copy(hidx_hbm.at[pl.ds(base, BPW)], hraw_v)
    pltpu.sync_copy(tidx_hbm.at[pl.ds(base, BPW)], traw_v)
    pltpu.sync_copy(ridx_hbm.at[pl.ds(base, BPW)], rraw_v)

    for raw, pix in ((hraw_v, hpix_v), (traw_v, tpix_v), (rraw_v, rpix_v)):
        for q in range(4):
            for k in range(8):
                v = raw[pl.ds(q * 128 + k * L, L)]
                pix[q, pl.ds(k * L, L)] = v - jnp.where(
                    v >= H, jnp.int32(H), jnp.int32(0))

    def fire(q, sem):
        dst = pl.ds(lax.rem(q, 2) * 128, 128)
        pltpu.async_copy(ent_hbm.at[hpix_v.at[q]], hbuf.at[dst], sem)
        pltpu.async_copy(ent_hbm.at[tpix_v.at[q]], tbuf.at[dst], sem)
        pltpu.async_copy(rel_hbm.at[rpix_v.at[q]], rbuf.at[dst], sem)

    def drain(sem):
        for buf in (hbuf, tbuf, rbuf):
            pltpu.make_async_copy(
                ent_hbm.at[pl.ds(0, 128)],
                buf.at[pl.ds(0, 128)], sem).wait()

    lane = jnp.arange(L, dtype=jnp.int32)
    masks = [lane == i for i in range(L)]

    GPQ = 8
    fire(0, sem0)

    def step(g, sq_acc):
        @pl.when(lax.rem(g, GPQ) == 0)
        def _():
            q = g // GPQ
            par = lax.rem(q, 2)

            @pl.when(q < 3)
            def _():
                @pl.when(par == 0)
                def _():
                    fire(q + 1, sem1)

                @pl.when(par == 1)
                def _():
                    fire(q + 1, sem0)

            @pl.when(par == 0)
            def _():
                drain(sem0)

            @pl.when(par == 1)
            def _():
                drain(sem1)

        hv = hraw_v[pl.ds(g * L, L)]
        tv = traw_v[pl.ds(g * L, L)]
        rv = rraw_v[pl.ds(g * L, L)]
        grow = lax.rem(g, 2 * GPQ) * L
        acc = jnp.zeros((L,), jnp.float32)
        for i in range(L):
            rowv = jnp.full((L,), grow + i, jnp.int32)
            hc = jnp.where(hv[i] >= H, D, 0) + lane
            tc = jnp.where(tv[i] >= H, D, 0) + lane
            rc = jnp.where(rv[i] >= H, D, 0) + lane
            racc = jnp.zeros((L,), jnp.float32)
            for k in range(CH):
                h = plsc.load_gather(hbuf, [rowv, hc + k * L])
                t = plsc.load_gather(tbuf, [rowv, tc + k * L])
                r = plsc.load_gather(rbuf, [rowv, rc + k * L])
                racc = racc + h * t * r
                sq_acc = sq_acc + h * h + t * t + r * r
            acc = jnp.where(masks[i], jnp.sum(racc), acc)
        score_v[pl.ds(g * L, L)] = -acc
        return sq_acc

    sq_acc = lax.fori_loop(0, GROUPS, step, jnp.zeros((L,), jnp.float32))

    part_v[...] = sq_acc
    pltpu.sync_copy(score_v, score_hbm.at[pl.ds(base, BPW)])
    pltpu.sync_copy(part_v, part_hbm.at[wid])


@jax.jit
def _dist_mult_sc(h_idx, r_idx, t_idx, ent_p, rel_p):
    mesh = plsc.VectorSubcoreMesh(core_axis_name="c", subcore_axis_name="s")
    call = functools.partial(
        pl.kernel,
        mesh=mesh,
        compiler_params=pltpu.CompilerParams(
            needs_layout_passes=False, use_tc_tiling_on_sc=True),
        out_type=[
            jax.ShapeDtypeStruct((B,), jnp.float32),
            jax.ShapeDtypeStruct((NW, L), jnp.float32),
        ],
        scratch_types=[
            pltpu.VMEM((BPW,), jnp.int32),
            pltpu.VMEM((BPW,), jnp.int32),
            pltpu.VMEM((BPW,), jnp.int32),
            pltpu.VMEM((4, 128), jnp.int32),
            pltpu.VMEM((4, 128), jnp.int32),
            pltpu.VMEM((4, 128), jnp.int32),
            pltpu.VMEM((HALF, 2 * D), jnp.float32),
            pltpu.VMEM((HALF, 2 * D), jnp.float32),
            pltpu.VMEM((HALF, 2 * D), jnp.float32),
            pltpu.VMEM((BPW,), jnp.float32),
            pltpu.VMEM((L,), jnp.float32),
            pltpu.SemaphoreType.DMA,
            pltpu.SemaphoreType.DMA,
        ],
    )(_dist_mult_body)
    return call(h_idx, r_idx, t_idx, ent_p, rel_p)


NE = 1000000
CW = 16384
NCHUNK = 31
H = NCHUNK * CW


def _repack_body(lo_ref, hi_ref, out_ref):
    eye = jnp.eye(D, dtype=jnp.float32)
    out_ref[:, 0:D] = jax.lax.dot_general(
        lo_ref[...], eye, (((0,), (0,)), ((), ())),
        preferred_element_type=jnp.float32)
    out_ref[:, D:2 * D] = jax.lax.dot_general(
        hi_ref[...], eye, (((0,), (0,)), ((), ())),
        preferred_element_type=jnp.float32)


def _repack(table_t):
    last_blk = (NE - 1) // CW
    return pl.pallas_call(
        _repack_body,
        grid=(NCHUNK,),
        in_specs=[
            pl.BlockSpec((D, CW), lambda i: (0, i)),
            pl.BlockSpec((D, CW),
                         lambda i: (0, jnp.minimum(i + NCHUNK, last_blk))),
        ],
        out_specs=pl.BlockSpec((CW, 2 * D), lambda i: (i, 0)),
        out_shape=jax.ShapeDtypeStruct((H, 2 * D), jnp.float32),
    )(table_t, table_t)


def kernel(batch_input, ent_embeddings, rel_embeddings):
    bi = batch_input.astype(jnp.int32)
    h_idx = bi[:, 0]
    r_idx = bi[:, 1]
    t_idx = bi[:, 2]
    ent_p = _repack(ent_embeddings.T)
    rel_p = _repack(rel_embeddings.T)
    score, part = _dist_mult_sc(h_idx, r_idx, t_idx, ent_p, rel_p)
    regul = jnp.sum(part) / jnp.float32(B * D)
    return (score, regul)
